# baseline (speedup 1.0000x reference)
import jax
import jax.numpy as jnp
from jax.experimental import pallas as pl
from jax.experimental.pallas import tpu as pltpu
from jax.experimental.pallas import tpu_sc as plsc

_VOCAB = 100000
_EMB = 128
_NPOS = 200
_HIDDEN = 50
_LIN1_STEPS = 10
_CHUNK = (_NPOS * _EMB) // _LIN1_STEPS
_VSUB = 8
_VLANE = _VOCAB // _VSUB
_GW = 128
_NPAD = 256


def _sc_gather(emb, idx2d):
    mesh = plsc.VectorSubcoreMesh(core_axis_name="core",
                                  subcore_axis_name="subcore")

    @pl.kernel(out_type=jax.ShapeDtypeStruct((_NPAD, _EMB), jnp.float32),
               mesh=mesh)
    def gather_kernel(emb_hbm, i_hbm, o_hbm):
        def body(i_vmem, o_vmem):
            pltpu.sync_copy(emb_hbm.at[i_vmem.at[0]], o_vmem)

        pltpu.emit_pipeline(
            body,
            grid=(_NPAD // _GW,),
            in_specs=[pl.BlockSpec((1, _GW), lambda i: (0, i))],
            out_specs=[pl.BlockSpec((_GW, _EMB), lambda i: (i, 0))],
            core_axis_name="subcore",
            dimension_semantics=(pltpu.PARALLEL,),
        )(i_hbm, o_hbm)

    return gather_kernel(emb, idx2d)


def _lin1_body(x_ref, w1_ref, b1_ref, o_ref):
    i = pl.program_id(0)
    part = jnp.dot(x_ref[0], w1_ref[0], preferred_element_type=jnp.float32)

    @pl.when(i == 0)
    def _():
        o_ref[...] = part

    @pl.when(i > 0)
    def _():
        o_ref[...] += part

    @pl.when(i == _LIN1_STEPS - 1)
    def _():
        o_ref[...] = jnp.maximum(o_ref[...] + b1_ref[...], 0.0)


def _lin1(xg, W1, b1):
    x3 = xg.reshape(_LIN1_STEPS, 1, _CHUNK)
    w3 = W1.reshape(_LIN1_STEPS, _CHUNK, _HIDDEN)
    return pl.pallas_call(
        _lin1_body,
        grid=(_LIN1_STEPS,),
        in_specs=[
            pl.BlockSpec((1, 1, _CHUNK), lambda i: (i, 0, 0)),
            pl.BlockSpec((1, _CHUNK, _HIDDEN), lambda i: (i, 0, 0)),
            pl.BlockSpec((1, _HIDDEN), lambda i: (0, 0)),
        ],
        out_specs=pl.BlockSpec((1, _HIDDEN), lambda i: (0, 0)),
        out_shape=jax.ShapeDtypeStruct((1, _HIDDEN), jnp.float32),
    )(x3, w3, b1.reshape(1, _HIDDEN))


def _lin2_body(s_ref, w2_ref, h_ref, b2_ref, o_ref, acc_ref):
    k = pl.program_id(0)
    w = h_ref[k]

    @pl.when(k == 0)
    def _():
        acc_ref[...] = w * w2_ref[0]

    @pl.when(k > 0)
    def _():
        acc_ref[...] += w * w2_ref[0]

    @pl.when(k == _HIDDEN - 1)
    def _():
        z = acc_ref[...] + b2_ref[0]
        m = jnp.max(z)
        lse = jnp.log(jnp.sum(jnp.exp(z - m))) + m
        o_ref[0] = z - lse


def _lin2_logsoftmax(s, h_vec, W2, b2):
    w2r = W2.reshape(_HIDDEN, _VSUB, _VLANE)
    b2r = b2.reshape(1, _VSUB, _VLANE)
    grid_spec = pltpu.PrefetchScalarGridSpec(
        num_scalar_prefetch=1,
        grid=(_HIDDEN,),
        in_specs=[
            pl.BlockSpec((1, _VSUB, _VLANE), lambda k, s_ref: (s_ref[k], 0, 0)),
            pl.BlockSpec(memory_space=pltpu.SMEM),
            pl.BlockSpec((1, _VSUB, _VLANE), lambda k, s_ref: (0, 0, 0)),
        ],
        out_specs=pl.BlockSpec((1, _VSUB, _VLANE), lambda k, s_ref: (0, 0, 0)),
        scratch_shapes=[pltpu.VMEM((_VSUB, _VLANE), jnp.float32)],
    )
    out = pl.pallas_call(
        _lin2_body,
        grid_spec=grid_spec,
        out_shape=jax.ShapeDtypeStruct((1, _VSUB, _VLANE), jnp.float32),
    )(s, w2r, h_vec, b2r)
    return out.reshape(1, _VOCAB)


def kernel(inp, emb, W1, b1, W2, b2):
    idx = inp.astype(jnp.int32)
    idx2d = jnp.zeros((1, _NPAD), jnp.int32).at[0, :_NPOS].set(idx)
    xg = _sc_gather(emb, idx2d)[:_NPOS]
    h = _lin1(xg, W1, b1)
    h_vec = h[0]
    iota = jnp.arange(_HIDDEN, dtype=jnp.int32)
    s = jax.lax.cummax(jnp.where(h_vec > 0, iota, 0))
    return _lin2_logsoftmax(s, h_vec, W2, b2)


# speedup vs baseline: 1.0533x; 1.0533x over previous
import jax
import jax.numpy as jnp
from jax.experimental import pallas as pl
from jax.experimental.pallas import tpu as pltpu
from jax.experimental.pallas import tpu_sc as plsc

_VOCAB = 100000
_EMB = 128
_NPOS = 200
_HIDDEN = 50
_LIN1_STEPS = 10
_CHUNK = (_NPOS * _EMB) // _LIN1_STEPS
_VSUB = 8
_VLANE = _VOCAB // _VSUB
_GW = 128
_NPAD = 256


def _sc_gather(emb, idx2d):
    mesh = plsc.VectorSubcoreMesh(core_axis_name="core",
                                  subcore_axis_name="subcore")

    @pl.kernel(out_type=jax.ShapeDtypeStruct((_NPAD, _EMB), jnp.float32),
               mesh=mesh)
    def gather_kernel(emb_hbm, i_hbm, o_hbm):
        def body(i_vmem, o_vmem):
            pltpu.sync_copy(emb_hbm.at[i_vmem.at[0]], o_vmem)

        pltpu.emit_pipeline(
            body,
            grid=(_NPAD // _GW,),
            in_specs=[pl.BlockSpec((1, _GW), lambda i: (0, i))],
            out_specs=[pl.BlockSpec((_GW, _EMB), lambda i: (i, 0))],
            core_axis_name=("core", "subcore"),
            dimension_semantics=(pltpu.PARALLEL,),
        )(i_hbm, o_hbm)

    return gather_kernel(emb, idx2d)


def _lin1_body(x_ref, w1_ref, b1_ref, h_ref, s_ref, acc_ref):
    i = pl.program_id(0)
    xc = x_ref[:, pl.ds(i * _CHUNK, _CHUNK)]
    part = jnp.dot(xc, w1_ref[0], preferred_element_type=jnp.float32)

    @pl.when(i == 0)
    def _():
        acc_ref[...] = part

    @pl.when(i > 0)
    def _():
        acc_ref[...] += part

    @pl.when(i == _LIN1_STEPS - 1)
    def _():
        h = jnp.maximum(acc_ref[...] + b1_ref[...], 0.0)
        h_ref[...] = h
        lane = jax.lax.broadcasted_iota(jnp.int32, (1, _HIDDEN), 1)
        s = jnp.where(h > 0, lane, 0)
        for sh in (1, 2, 4, 8, 16, 32):
            rolled = pltpu.roll(s, sh, axis=1)
            s = jnp.maximum(s, jnp.where(lane >= sh, rolled, 0))
        s_ref[...] = s


def _lin1(xflat, W1, b1):
    w3 = W1.reshape(_LIN1_STEPS, _CHUNK, _HIDDEN)
    return pl.pallas_call(
        _lin1_body,
        grid=(_LIN1_STEPS,),
        in_specs=[
            pl.BlockSpec((1, xflat.shape[1]), lambda i: (0, 0)),
            pl.BlockSpec((1, _CHUNK, _HIDDEN), lambda i: (i, 0, 0)),
            pl.BlockSpec((1, _HIDDEN), lambda i: (0, 0)),
        ],
        out_specs=[
            pl.BlockSpec((1, _HIDDEN), lambda i: (0, 0)),
            pl.BlockSpec((1, _HIDDEN), lambda i: (0, 0)),
        ],
        out_shape=[
            jax.ShapeDtypeStruct((1, _HIDDEN), jnp.float32),
            jax.ShapeDtypeStruct((1, _HIDDEN), jnp.int32),
        ],
        scratch_shapes=[pltpu.VMEM((1, _HIDDEN), jnp.float32)],
    )(xflat, w3, b1.reshape(1, _HIDDEN))


def _lin2_body(s_ref, w2_ref, h_ref, b2_ref, o_ref, acc_ref):
    k = pl.program_id(0)
    w = h_ref[k]

    @pl.when(k == 0)
    def _():
        acc_ref[...] = w * w2_ref[0]

    @pl.when((k > 0) & (w > 0))
    def _():
        acc_ref[...] += w * w2_ref[0]

    @pl.when(k == _HIDDEN - 1)
    def _():
        z = acc_ref[...] + b2_ref[0]
        m = jnp.max(z)
        lse = jnp.log(jnp.sum(jnp.exp(z - m))) + m
        o_ref[0] = z - lse


def _lin2_logsoftmax(s, h_vec, W2, b2):
    w2r = W2.reshape(_HIDDEN, _VSUB, _VLANE)
    b2r = b2.reshape(1, _VSUB, _VLANE)
    grid_spec = pltpu.PrefetchScalarGridSpec(
        num_scalar_prefetch=1,
        grid=(_HIDDEN,),
        in_specs=[
            pl.BlockSpec((1, _VSUB, _VLANE), lambda k, s_ref: (s_ref[k], 0, 0)),
            pl.BlockSpec(memory_space=pltpu.SMEM),
            pl.BlockSpec((1, _VSUB, _VLANE), lambda k, s_ref: (0, 0, 0)),
        ],
        out_specs=pl.BlockSpec((1, _VSUB, _VLANE), lambda k, s_ref: (0, 0, 0)),
        scratch_shapes=[pltpu.VMEM((_VSUB, _VLANE), jnp.float32)],
    )
    out = pl.pallas_call(
        _lin2_body,
        grid_spec=grid_spec,
        out_shape=jax.ShapeDtypeStruct((1, _VSUB, _VLANE), jnp.float32),
    )(s, w2r, h_vec, b2r)
    return out.reshape(1, _VOCAB)


def kernel(inp, emb, W1, b1, W2, b2):
    idx = inp.astype(jnp.int32)
    idx2d = jnp.zeros((1, _NPAD), jnp.int32).at[0, :_NPOS].set(idx)
    xg = _sc_gather(emb, idx2d)
    xflat = xg.reshape(1, _NPAD * _EMB)
    h, s = _lin1(xflat, W1, b1)
    return _lin2_logsoftmax(s.reshape(_HIDDEN), h.reshape(_HIDDEN), W2, b2)


# speedup vs baseline: 1.8197x; 1.7276x over previous
import jax
import jax.numpy as jnp
from jax.experimental import pallas as pl
from jax.experimental.pallas import tpu as pltpu
from jax.experimental.pallas import tpu_sc as plsc

_VOCAB = 100000
_EMB = 128
_NPOS = 200
_HIDDEN = 50
_LIN1_STEPS = 2
_CHUNK = (_NPOS * _EMB) // _LIN1_STEPS
_VBLK = 12800
_NBLK = 8
_GW = 128
_NPAD = 256


def _sc_gather(emb, idx2d):
    mesh = plsc.VectorSubcoreMesh(core_axis_name="core",
                                  subcore_axis_name="subcore")

    @pl.kernel(out_type=jax.ShapeDtypeStruct((_NPAD, _EMB), jnp.float32),
               mesh=mesh)
    def gather_kernel(emb_hbm, i_hbm, o_hbm):
        def body(i_vmem, o_vmem):
            pltpu.sync_copy(emb_hbm.at[i_vmem.at[0]], o_vmem)

        pltpu.emit_pipeline(
            body,
            grid=(_NPAD // _GW,),
            in_specs=[pl.BlockSpec((1, _GW), lambda i: (0, i))],
            out_specs=[pl.BlockSpec((_GW, _EMB), lambda i: (i, 0))],
            core_axis_name=("core", "subcore"),
            dimension_semantics=(pltpu.PARALLEL,),
        )(i_hbm, o_hbm)

    return gather_kernel(emb, idx2d)


def _lin1_body(x_ref, w1_ref, b1_ref, h_ref):
    i = pl.program_id(0)
    xc = x_ref[:, pl.ds(i * _CHUNK, _CHUNK)]
    part = jnp.dot(xc, w1_ref[0], preferred_element_type=jnp.float32)

    @pl.when(i == 0)
    def _():
        h_ref[...] = part

    @pl.when(i == _LIN1_STEPS - 1)
    def _():
        h_ref[...] = jnp.maximum(h_ref[...] + part + b1_ref[...], 0.0)


def _lin1(xflat, W1, b1):
    w3 = W1.reshape(_LIN1_STEPS, _CHUNK, _HIDDEN)
    return pl.pallas_call(
        _lin1_body,
        grid=(_LIN1_STEPS,),
        in_specs=[
            pl.BlockSpec((1, xflat.shape[1]), lambda i: (0, 0)),
            pl.BlockSpec((1, _CHUNK, _HIDDEN), lambda i: (i, 0, 0)),
            pl.BlockSpec((1, _HIDDEN), lambda i: (0, 0)),
        ],
        out_specs=pl.BlockSpec((1, _HIDDEN), lambda i: (0, 0)),
        out_shape=jax.ShapeDtypeStruct((1, _HIDDEN), jnp.float32),
    )(xflat, w3, b1.reshape(1, _HIDDEN))


def _lin2_body(h_ref, w2_ref, b2_ref, o_ref, acc_ref, m_ref, l_ref):
    k = pl.program_id(0)

    @pl.when(k < _NBLK)
    def _():
        zp = jnp.dot(h_ref[...], w2_ref[...],
                     preferred_element_type=jnp.float32) + b2_ref[...]
        lane = jax.lax.broadcasted_iota(jnp.int32, (1, _VBLK), 1)
        valid = (k * _VBLK + lane) < _VOCAB
        zm = jnp.where(valid, zp, -jnp.inf)
        mk = jnp.max(zm)
        lk = jnp.sum(jnp.where(valid, jnp.exp(zm - mk), 0.0))
        for j in range(_NBLK):
            @pl.when(k == j)
            def _():
                acc_ref[0:1, pl.ds(j * _VBLK, _VBLK)] = zp

        @pl.when(k == 0)
        def _():
            m_ref[0] = mk
            l_ref[0] = lk

        @pl.when(k > 0)
        def _():
            m_old = m_ref[0]
            m_new = jnp.maximum(m_old, mk)
            l_ref[0] = (l_ref[0] * jnp.exp(m_old - m_new)
                        + lk * jnp.exp(mk - m_new))
            m_ref[0] = m_new

    @pl.when(k == _NBLK)
    def _():
        lse = jnp.log(l_ref[0]) + m_ref[0]
        o_ref[...] = acc_ref[0:1, pl.ds(0, _VOCAB)] - lse


def _lin2_logsoftmax(h, W2, b2):
    grid_spec = pltpu.PrefetchScalarGridSpec(
        num_scalar_prefetch=0,
        grid=(_NBLK + 1,),
        in_specs=[
            pl.BlockSpec((1, _HIDDEN), lambda k: (0, 0)),
            pl.BlockSpec((_HIDDEN, _VBLK),
                         lambda k: (0, jnp.minimum(k, _NBLK - 1))),
            pl.BlockSpec((1, _VBLK),
                         lambda k: (0, jnp.minimum(k, _NBLK - 1))),
        ],
        out_specs=pl.BlockSpec((1, _VOCAB), lambda k: (0, 0)),
        scratch_shapes=[
            pltpu.VMEM((1, _NBLK * _VBLK), jnp.float32),
            pltpu.SMEM((1,), jnp.float32),
            pltpu.SMEM((1,), jnp.float32),
        ],
    )
    return pl.pallas_call(
        _lin2_body,
        grid_spec=grid_spec,
        out_shape=jax.ShapeDtypeStruct((1, _VOCAB), jnp.float32),
    )(h, W2, b2.reshape(1, _VOCAB))


def kernel(inp, emb, W1, b1, W2, b2):
    idx = inp.astype(jnp.int32)
    idx2d = jnp.zeros((1, _NPAD), jnp.int32).at[0, :_NPOS].set(idx)
    xg = _sc_gather(emb, idx2d)
    xflat = xg.reshape(1, _NPAD * _EMB)
    h = _lin1(xflat, W1, b1)
    return _lin2_logsoftmax(h, W2, b2)
